# baseline (device time: 11685 ns/iter reference)
import jax
import jax.numpy as jnp
from jax import lax
from jax.experimental import pallas as pl
from jax.experimental.pallas import tpu as pltpu

Z = 2
NCHUNK = 8


def kernel(x):
    m, n = x.shape
    half = n // Z

    def body(x_ref, out_ref, xv, loc_f32, send_buf, loc_bf16,
             in_sems, out_copy_sem, send_sems, recv_sems):
        my_x = lax.axis_index("x")
        my_y = lax.axis_index("y")
        my_z = lax.axis_index("z")
        pz = 1 - my_z

        barrier_sem = pltpu.get_barrier_semaphore()
        pl.semaphore_signal(
            barrier_sem, inc=1,
            device_id=(my_x, my_y, pz),
            device_id_type=pl.DeviceIdType.MESH,
        )

        rows = m // NCHUNK
        in_dmas = []
        for k in range(NCHUNK):
            dma = pltpu.make_async_copy(
                x_ref.at[pl.ds(k * rows, rows), pl.ds(pz * half, half)],
                xv.at[pl.ds(k * rows, rows), :],
                in_sems.at[k],
            )
            dma.start()
            in_dmas.append(dma)

        pl.semaphore_wait(barrier_sem, 1)

        rdmas = []
        for k in range(NCHUNK):
            in_dmas[k].wait()
            send_buf[pl.ds(k * rows, rows), :] = (
                xv[pl.ds(k * rows, rows), :].astype(jnp.bfloat16)
            )
            rdma = pltpu.make_async_remote_copy(
                src_ref=send_buf.at[pl.ds(k * rows, rows), :],
                dst_ref=out_ref.at[pl.ds(my_z * m + k * rows, rows), :],
                send_sem=send_sems.at[k],
                recv_sem=recv_sems.at[k],
                device_id=(my_x, my_y, pz),
                device_id_type=pl.DeviceIdType.MESH,
            )
            rdma.start()
            rdmas.append(rdma)

        dma_loc_half = pltpu.make_async_copy(
            x_ref.at[:, pl.ds(my_z * half, half)], loc_f32, in_sems.at[NCHUNK]
        )
        dma_loc_half.start()
        dma_loc_half.wait()
        loc_bf16[:, :] = loc_f32[:, :].astype(jnp.bfloat16)
        out_copy = pltpu.make_async_copy(
            loc_bf16, out_ref.at[pl.ds(my_z * m, m), :], out_copy_sem
        )
        out_copy.start()

        out_copy.wait()
        for rdma in rdmas:
            rdma.wait()

    x = pltpu.with_memory_space_constraint(x, pltpu.HBM)
    return pl.pallas_call(
        body,
        out_shape=jax.ShapeDtypeStruct((Z * m, half), jnp.bfloat16),
        in_specs=[pl.BlockSpec(memory_space=pltpu.HBM)],
        out_specs=pl.BlockSpec(memory_space=pltpu.HBM),
        scratch_shapes=[
            pltpu.VMEM((m, half), x.dtype),
            pltpu.VMEM((m, half), x.dtype),
            pltpu.VMEM((m, half), jnp.bfloat16),
            pltpu.VMEM((m, half), jnp.bfloat16),
            pltpu.SemaphoreType.DMA((NCHUNK + 1,)),
            pltpu.SemaphoreType.DMA,
            pltpu.SemaphoreType.DMA((NCHUNK,)),
            pltpu.SemaphoreType.DMA((NCHUNK,)),
        ],
        compiler_params=pltpu.CompilerParams(collective_id=0),
    )(x)


# device time: 11610 ns/iter; 1.0065x vs baseline; 1.0065x over previous
import jax
import jax.numpy as jnp
from jax import lax
from jax.experimental import pallas as pl
from jax.experimental.pallas import tpu as pltpu

Z = 2
NCHUNK = 4


def kernel(x):
    m, n = x.shape
    half = n // Z

    def body(x_ref, out_ref, xv, loc_f32, send_buf, loc_bf16,
             in_sems, out_copy_sem, send_sems, recv_sems):
        my_x = lax.axis_index("x")
        my_y = lax.axis_index("y")
        my_z = lax.axis_index("z")
        pz = 1 - my_z

        barrier_sem = pltpu.get_barrier_semaphore()
        pl.semaphore_signal(
            barrier_sem, inc=1,
            device_id=(my_x, my_y, pz),
            device_id_type=pl.DeviceIdType.MESH,
        )

        rows = m // NCHUNK
        in_dmas = []
        for k in range(NCHUNK):
            dma = pltpu.make_async_copy(
                x_ref.at[pl.ds(k * rows, rows), pl.ds(pz * half, half)],
                xv.at[pl.ds(k * rows, rows), :],
                in_sems.at[k],
            )
            dma.start()
            in_dmas.append(dma)

        pl.semaphore_wait(barrier_sem, 1)

        rdmas = []
        for k in range(NCHUNK):
            in_dmas[k].wait()
            send_buf[pl.ds(k * rows, rows), :] = (
                xv[pl.ds(k * rows, rows), :].astype(jnp.bfloat16)
            )
            rdma = pltpu.make_async_remote_copy(
                src_ref=send_buf.at[pl.ds(k * rows, rows), :],
                dst_ref=out_ref.at[pl.ds(my_z * m + k * rows, rows), :],
                send_sem=send_sems.at[k],
                recv_sem=recv_sems.at[k],
                device_id=(my_x, my_y, pz),
                device_id_type=pl.DeviceIdType.MESH,
            )
            rdma.start()
            rdmas.append(rdma)

        dma_loc_half = pltpu.make_async_copy(
            x_ref.at[:, pl.ds(my_z * half, half)], loc_f32, in_sems.at[NCHUNK]
        )
        dma_loc_half.start()
        dma_loc_half.wait()
        loc_bf16[:, :] = loc_f32[:, :].astype(jnp.bfloat16)
        out_copy = pltpu.make_async_copy(
            loc_bf16, out_ref.at[pl.ds(my_z * m, m), :], out_copy_sem
        )
        out_copy.start()

        out_copy.wait()
        for rdma in rdmas:
            rdma.wait()

    x = pltpu.with_memory_space_constraint(x, pltpu.HBM)
    return pl.pallas_call(
        body,
        out_shape=jax.ShapeDtypeStruct((Z * m, half), jnp.bfloat16),
        in_specs=[pl.BlockSpec(memory_space=pltpu.HBM)],
        out_specs=pl.BlockSpec(memory_space=pltpu.HBM),
        scratch_shapes=[
            pltpu.VMEM((m, half), x.dtype),
            pltpu.VMEM((m, half), x.dtype),
            pltpu.VMEM((m, half), jnp.bfloat16),
            pltpu.VMEM((m, half), jnp.bfloat16),
            pltpu.SemaphoreType.DMA((NCHUNK + 1,)),
            pltpu.SemaphoreType.DMA,
            pltpu.SemaphoreType.DMA((NCHUNK,)),
            pltpu.SemaphoreType.DMA((NCHUNK,)),
        ],
        compiler_params=pltpu.CompilerParams(collective_id=0),
    )(x)
